# baseline (speedup 1.0000x reference)
import functools

import jax
import jax.numpy as jnp
from jax import lax
from jax.experimental import pallas as pl
from jax.experimental.pallas import tpu as pltpu
from jax.experimental.pallas import tpu_sc as plsc

N_FIELDS = 26
VOCAB = 100000
EMB = 32
B = 4096
D0 = N_FIELDS * EMB
H1 = 256
H2 = 128
NCLS = 10
EPS = 1e-5

NC = 2
NS = 16
NW = NC * NS

NGRP = 2
FG = N_FIELDS // NGRP
ROWS_G = B * FG
CHUNK = 128
CHUNKS_PER_W = ROWS_G // (NW * CHUNK)
IDS = CHUNK * EMB


def _sc_gather(table1d, base_ids):
    mesh = plsc.VectorSubcoreMesh(core_axis_name="c", subcore_axis_name="s")

    @functools.partial(
        pl.kernel,
        mesh=mesh,
        compiler_params=pltpu.CompilerParams(use_tc_tiling_on_sc=False),
        out_type=jax.ShapeDtypeStruct((ROWS_G * EMB,), jnp.float32),
        scratch_types=[
            pltpu.VMEM((CHUNKS_PER_W * CHUNK,), jnp.int32),
            pltpu.VMEM((2, IDS), jnp.int32),
            pltpu.VMEM((2, IDS), jnp.float32),
            pltpu.SemaphoreType.DMA,
            pltpu.SemaphoreType.DMA,
        ],
    )
    def k(table_hbm, base_hbm, out_hbm, base_v, ids_v, data_v, gsem, osem):
        wid = lax.axis_index("s") * NC + lax.axis_index("c")
        wbase = wid * CHUNKS_PER_W * CHUNK
        pltpu.sync_copy(
            base_hbm.at[pl.ds(wbase, CHUNKS_PER_W * CHUNK)], base_v)

        lane = lax.iota(jnp.int32, 16)
        evec_lo = lane * VOCAB
        evec_hi = (lane + 16) * VOCAB

        def build_ids(j, buf):
            def grp(g, _):
                b16 = base_v[pl.ds(j * CHUNK + g * 16, 16)]
                for rr in range(16):
                    bs = lax.gather(
                        b16, jnp.full((16, 1), rr, jnp.int32),
                        lax.GatherDimensionNumbers(
                            offset_dims=(), collapsed_slice_dims=(0,),
                            start_index_map=(0,)),
                        slice_sizes=(1,),
                        mode=lax.GatherScatterMode.PROMISE_IN_BOUNDS)
                    pos = (g * 16 + rr) * EMB
                    ids_v[buf, pl.ds(pos, 16)] = bs + evec_lo
                    ids_v[buf, pl.ds(pos + 16, 16)] = bs + evec_hi
                return ()

            lax.fori_loop(0, CHUNK // 16, grp, ())

        def fire(j, buf):
            def q_step(q, _):
                off = pl.multiple_of(q * CHUNK, 8)
                pltpu.async_copy(
                    table_hbm.at[ids_v.at[buf].at[pl.ds(off, CHUNK)]],
                    data_v.at[buf].at[pl.ds(off, CHUNK)],
                    gsem)
                return ()

            lax.fori_loop(0, EMB, q_step, ())

        def drain(buf):
            def q_step(q, _):
                pltpu.make_async_copy(
                    table_hbm.at[ids_v.at[buf].at[pl.ds(0, CHUNK)]],
                    data_v.at[buf].at[pl.ds(0, CHUNK)],
                    gsem).wait()
                return ()

            lax.fori_loop(0, EMB, q_step, ())

        def flush(j, buf):
            off = pl.multiple_of((wbase + j * CHUNK) * EMB, 8)
            pltpu.async_copy(
                data_v.at[buf], out_hbm.at[pl.ds(off, IDS)], osem).wait()

        build_ids(0, 0)
        fire(0, 0)

        def step(j, _):
            build_ids(2 * j + 1, 1)
            fire(2 * j + 1, 1)
            drain(0)
            flush(2 * j, 0)
            build_ids(2 * j + 2, 0)
            fire(2 * j + 2, 0)
            drain(1)
            flush(2 * j + 1, 1)
            return ()

        lax.fori_loop(0, (CHUNKS_PER_W - 1) // 2, step, ())

        drain(0)
        flush(CHUNKS_PER_W - 1, 0)

    return k(table1d, base_ids)


def _mlp_body(z0_ref, z1_ref, g0, b0, w1, b1, g1, bb1, w2, b2, g2, bb2, w3,
              b3, out_ref):
    def bn(x, g, b):
        mu = jnp.mean(x, axis=0, keepdims=True)
        var = jnp.mean((x - mu) * (x - mu), axis=0, keepdims=True)
        return (x - mu) * lax.rsqrt(var + EPS) * g[...] + b[...]

    half = FG * EMB
    z0 = bn(z0_ref[...], g0[:, :half], b0[:, :half])
    z1 = bn(z1_ref[...], g0[:, half:], b0[:, half:])
    h = (jnp.dot(z0, w1[:half, :], preferred_element_type=jnp.float32,
                 precision=lax.Precision.HIGHEST) +
         jnp.dot(z1, w1[half:, :], preferred_element_type=jnp.float32,
                 precision=lax.Precision.HIGHEST))
    h = jnp.maximum(h + b1[...], 0.0)
    h = bn(h, g1, bb1)
    h = jnp.dot(h, w2[...], preferred_element_type=jnp.float32,
                precision=lax.Precision.HIGHEST)
    h = jnp.maximum(h + b2[...], 0.0)
    h = bn(h, g2, bb2)
    out = jnp.dot(h, w3[...], preferred_element_type=jnp.float32,
                  precision=lax.Precision.HIGHEST)
    out_ref[...] = out + b3[...]


def kernel(x_cat, tables, bn0_g, bn0_b, W1, b1, bn1_g, bn1_b, W2, b2, bn2_g, bn2_b, W3, b3):
    x32 = x_cat.astype(jnp.int32)
    offs = (jnp.arange(FG, dtype=jnp.int32) * (EMB * VOCAB))[None, :]

    zs = []
    for g in range(NGRP):
        tgrp = lax.slice_in_dim(tables, g * FG, (g + 1) * FG, axis=0)
        table1d = tgrp.transpose(0, 2, 1).reshape(FG * EMB * VOCAB)
        base_ids = (x32[:, g * FG:(g + 1) * FG] + offs).reshape(ROWS_G)
        zs.append(_sc_gather(table1d, base_ids).reshape(B, FG * EMB))

    out = pl.pallas_call(
        _mlp_body,
        out_shape=jax.ShapeDtypeStruct((B, NCLS), jnp.float32),
    )(
        zs[0], zs[1],
        bn0_g.reshape(1, D0), bn0_b.reshape(1, D0),
        W1, b1.reshape(1, H1), bn1_g.reshape(1, H1), bn1_b.reshape(1, H1),
        W2, b2.reshape(1, H2), bn2_g.reshape(1, H2), bn2_b.reshape(1, H2),
        W3, b3.reshape(1, NCLS),
    )
    return out


# speedup vs baseline: 1.2559x; 1.2559x over previous
import functools

import jax
import jax.numpy as jnp
from jax import lax
from jax.experimental import pallas as pl
from jax.experimental.pallas import tpu as pltpu
from jax.experimental.pallas import tpu_sc as plsc

N_FIELDS = 26
VOCAB = 100000
EMB = 32
B = 4096
D0 = N_FIELDS * EMB
H1 = 256
H2 = 128
NCLS = 10
EPS = 1e-5

NC = 2
NS = 16
NW = NC * NS

ROWS = B * N_FIELDS
CHUNK = 256
CHUNKS_PER_W = ROWS // (NW * CHUNK)
IDS = CHUNK * EMB


def _sc_gather(table1d, base_ids):
    mesh = plsc.VectorSubcoreMesh(core_axis_name="c", subcore_axis_name="s")

    @functools.partial(
        pl.kernel,
        mesh=mesh,
        compiler_params=pltpu.CompilerParams(use_tc_tiling_on_sc=False),
        out_type=jax.ShapeDtypeStruct((ROWS * EMB,), jnp.float32),
        scratch_types=[
            pltpu.VMEM((CHUNKS_PER_W * CHUNK,), jnp.int32),
            pltpu.VMEM((2, IDS), jnp.int32),
            pltpu.VMEM((2, IDS), jnp.float32),
            pltpu.SemaphoreType.DMA,
            pltpu.SemaphoreType.DMA,
        ],
    )
    def k(table_hbm, base_hbm, out_hbm, base_v, ids_v, data_v, gsem, osem):
        wid = lax.axis_index("s") * NC + lax.axis_index("c")
        wbase = wid * CHUNKS_PER_W * CHUNK
        pltpu.sync_copy(
            base_hbm.at[pl.ds(wbase, CHUNKS_PER_W * CHUNK)], base_v)

        lane = lax.iota(jnp.int32, 16)
        evec_lo = lane * VOCAB
        evec_hi = (lane + 16) * VOCAB

        def build_ids(j, buf):
            def grp(g, _):
                b16 = base_v[pl.ds(j * CHUNK + g * 16, 16)]
                for rr in range(16):
                    bs = lax.gather(
                        b16, jnp.full((16, 1), rr, jnp.int32),
                        lax.GatherDimensionNumbers(
                            offset_dims=(), collapsed_slice_dims=(0,),
                            start_index_map=(0,)),
                        slice_sizes=(1,),
                        mode=lax.GatherScatterMode.PROMISE_IN_BOUNDS)
                    pos = (g * 16 + rr) * EMB
                    ids_v[buf, pl.ds(pos, 16)] = bs + evec_lo
                    ids_v[buf, pl.ds(pos + 16, 16)] = bs + evec_hi
                return ()

            lax.fori_loop(0, CHUNK // 16, grp, ())

        def fire(j, buf):
            def q_step(q, _):
                off = pl.multiple_of(q * CHUNK, 8)
                pltpu.async_copy(
                    table_hbm.at[ids_v.at[buf].at[pl.ds(off, CHUNK)]],
                    data_v.at[buf].at[pl.ds(off, CHUNK)],
                    gsem)
                return ()

            lax.fori_loop(0, EMB, q_step, ())

        def drain(buf):
            def q_step(q, _):
                pltpu.make_async_copy(
                    table_hbm.at[ids_v.at[buf].at[pl.ds(0, CHUNK)]],
                    data_v.at[buf].at[pl.ds(0, CHUNK)],
                    gsem).wait()
                return ()

            lax.fori_loop(0, EMB, q_step, ())

        def flush(j, buf):
            off = pl.multiple_of((wbase + j * CHUNK) * EMB, 8)
            pltpu.async_copy(
                data_v.at[buf], out_hbm.at[pl.ds(off, IDS)], osem).wait()

        build_ids(0, 0)
        fire(0, 0)

        def step(j, _):
            build_ids(2 * j + 1, 1)
            fire(2 * j + 1, 1)
            drain(0)
            flush(2 * j, 0)
            build_ids(2 * j + 2, 0)
            fire(2 * j + 2, 0)
            drain(1)
            flush(2 * j + 1, 1)
            return ()

        lax.fori_loop(0, (CHUNKS_PER_W - 1) // 2, step, ())

        drain(0)
        flush(CHUNKS_PER_W - 1, 0)

    return k(table1d, base_ids)


def _mlp_body(z_ref, g0, b0, w1, b1, g1, bb1, w2, b2, g2, bb2, w3, b3, out_ref):
    def bn(x, g, b):
        mu = jnp.mean(x, axis=0, keepdims=True)
        var = jnp.mean((x - mu) * (x - mu), axis=0, keepdims=True)
        return (x - mu) * lax.rsqrt(var + EPS) * g[...] + b[...]

    z = bn(z_ref[...], g0, b0)
    h = jnp.dot(z, w1[...], preferred_element_type=jnp.float32,
                precision=lax.Precision.HIGHEST)
    h = jnp.maximum(h + b1[...], 0.0)
    h = bn(h, g1, bb1)
    h = jnp.dot(h, w2[...], preferred_element_type=jnp.float32,
                precision=lax.Precision.HIGHEST)
    h = jnp.maximum(h + b2[...], 0.0)
    h = bn(h, g2, bb2)
    out = jnp.dot(h, w3[...], preferred_element_type=jnp.float32,
                  precision=lax.Precision.HIGHEST)
    out_ref[...] = out + b3[...]


def kernel(x_cat, tables, bn0_g, bn0_b, W1, b1, bn1_g, bn1_b, W2, b2, bn2_g, bn2_b, W3, b3):
    table1d = tables.transpose(0, 2, 1).reshape(N_FIELDS * EMB * VOCAB)

    offs = (jnp.arange(N_FIELDS, dtype=jnp.int32) * (EMB * VOCAB))[None, :]
    base_ids = (x_cat.astype(jnp.int32) + offs).reshape(ROWS)

    flat = _sc_gather(table1d, base_ids)
    z = flat.reshape(B, D0)

    out = pl.pallas_call(
        _mlp_body,
        out_shape=jax.ShapeDtypeStruct((B, NCLS), jnp.float32),
    )(
        z,
        bn0_g.reshape(1, D0), bn0_b.reshape(1, D0),
        W1, b1.reshape(1, H1), bn1_g.reshape(1, H1), bn1_b.reshape(1, H1),
        W2, b2.reshape(1, H2), bn2_g.reshape(1, H2), bn2_b.reshape(1, H2),
        W3, b3.reshape(1, NCLS),
    )
    return out
